# baseline (speedup 1.0000x reference)
import functools

import jax
import jax.numpy as jnp
from jax import lax
from jax.experimental import pallas as pl
from jax.experimental.pallas import tpu as pltpu
from jax.experimental.pallas import tpu_sc as plsc

NC = 2
NS = 16
LANES = 16
CHUNK = 64
NBUF = 4
WGRP = 16


def _sc_partials(x, src, dst, w):
    n, d = x.shape
    e = src.shape[0]
    nw = NC * NS
    e_per_w = e // nw
    assert e_per_w * nw == e and e_per_w % CHUNK == 0 and CHUNK % 8 == 0
    n_chunks = e_per_w // CHUNK
    assert n_chunks % NBUF == 0
    n_rounds = n_chunks // NBUF
    span = 40
    n_spans = n // span
    assert n % span == 0
    span_rounds = -(-n_spans // NS)
    nvec = d // LANES
    egrp = CHUNK // WGRP

    mesh = plsc.VectorSubcoreMesh(core_axis_name="c", subcore_axis_name="s")

    @functools.partial(
        pl.kernel,
        out_type=jax.ShapeDtypeStruct((NC, n, d), jnp.float32),
        mesh=mesh,
        scratch_types=[
            pltpu.VMEM((NBUF, CHUNK, d), jnp.float32),
            pltpu.VMEM((NBUF, CHUNK), jnp.int32),
            pltpu.VMEM((NBUF, CHUNK), jnp.int32),
            pltpu.VMEM((NBUF, CHUNK), jnp.float32),
            pltpu.VMEM_SHARED((n, d), jnp.float32),
            pltpu.SemaphoreType.DMA((NBUF,)),
            pltpu.SemaphoreType.DMA((NBUF,)),
            pltpu.SemaphoreType.DMA((NBUF,)),
            pltpu.SemaphoreType.DMA((NBUF,)),
            pltpu.SemaphoreType.DMA((NBUF,)),
        ],
    )
    def sc_kernel(x_hbm, src_hbm, dst_hbm, w_hbm, out_hbm, rows, srcb, dstb,
                  wb, acc, sem_g, sem_s, sem_src, sem_w, sem_d):
        cid = lax.axis_index("c")
        sid = lax.axis_index("s")
        wid = sid * NC + cid
        ebase = wid * e_per_w

        def src_copy(j, b):
            return pltpu.make_async_copy(
                src_hbm.at[pl.ds(ebase + j * CHUNK, CHUNK)], srcb.at[b],
                sem_src.at[b])

        def w_copy(j, b):
            return pltpu.make_async_copy(
                w_hbm.at[pl.ds(ebase + j * CHUNK, CHUNK)], wb.at[b],
                sem_w.at[b])

        def dst_copy(j, b):
            return pltpu.make_async_copy(
                dst_hbm.at[pl.ds(ebase + j * CHUNK, CHUNK)], dstb.at[b],
                sem_d.at[b])

        def gather_copy(j, b):
            return pltpu.make_async_copy(
                x_hbm.at[srcb.at[b]], rows.at[b], sem_g.at[b])

        def scale_chunk(b):
            rows_b = rows.at[b]
            for g in range(egrp):
                w16 = wb[b, pl.ds(WGRP * g, LANES)]
                for ee in range(WGRP):
                    wsplat = jnp.full((LANES,), w16[ee])
                    i = WGRP * g + ee
                    for f in range(nvec):
                        sl = (i, pl.ds(LANES * f, LANES))
                        rows_b[sl] = rows_b[sl] * wsplat

        for c in range(4):
            src_copy(c, c).start()
            w_copy(c, c).start()
        for c in range(2):
            dst_copy(c, c).start()
        for c in range(2):
            src_copy(c, c).wait()
            gather_copy(c, c).start()

        zeros = jnp.zeros((LANES,), jnp.float32)

        def zero_row(i, carry):
            for f in range(nvec):
                rows[NBUF - 1, i, pl.ds(LANES * f, LANES)] = zeros
            return carry

        lax.fori_loop(0, span, zero_row, 0)

        for j in range(span_rounds):
            c = j * NS + sid

            @pl.when(c < n_spans)
            def _():
                pltpu.async_copy(rows.at[NBUF - 1, pl.ds(0, span)],
                                 acc.at[pl.ds(c * span, span)],
                                 sem_s.at[0])
        for j in range(span_rounds):
            c = j * NS + sid

            @pl.when(c < n_spans)
            def _():
                pltpu.make_async_copy(
                    rows.at[NBUF - 1, pl.ds(0, span)],
                    acc.at[pl.ds(c * span, span)], sem_s.at[0]).wait()
        plsc.subcore_barrier()

        def round_body(q, carry):
            for k in range(NBUF):
                j = q * NBUF + k
                b2 = (k + 2) % NBUF

                @pl.when(j >= 2)
                def _():
                    pltpu.make_async_copy(
                        rows.at[b2], acc.at[dstb.at[b2]], sem_s.at[b2]).wait()

                @pl.when(j + 2 < n_chunks)
                def _():
                    dst_copy(j + 2, b2).start()
                    src_copy(j + 2, b2).wait()
                    gather_copy(j + 2, b2).start()

                gather_copy(j, k).wait()

                @pl.when(j + 4 < n_chunks)
                def _():
                    src_copy(j + 4, k).start()

                w_copy(j, k).wait()
                scale_chunk(k)

                @pl.when(j + 4 < n_chunks)
                def _():
                    w_copy(j + 4, k).start()

                dst_copy(j, k).wait()
                pltpu.async_copy(rows.at[k], acc.at[dstb.at[k]], sem_s.at[k],
                                 add=True)
            return carry

        lax.fori_loop(0, n_rounds, round_body, 0)
        for c in range(n_chunks - 2, n_chunks):
            k = c % NBUF
            pltpu.make_async_copy(
                rows.at[k], acc.at[dstb.at[k]], sem_s.at[k]).wait()

        plsc.subcore_barrier()

        def stage_a(c, b):
            return pltpu.make_async_copy(
                acc.at[pl.ds(c * span, span)], rows.at[b, pl.ds(0, span)],
                sem_g.at[b])

        def stage_b(c, b):
            return pltpu.make_async_copy(
                rows.at[b, pl.ds(0, span)],
                out_hbm.at[cid, pl.ds(c * span, span)], sem_s.at[b])

        for j in range(span_rounds):
            c = j * NS + sid
            b = j % 2

            @pl.when(c < n_spans)
            def _():
                if j >= 2:
                    cprev = (j - 2) * NS + sid
                    stage_b(cprev, b).wait()
                stage_a(c, b).start()
                stage_a(c, b).wait()
                stage_b(c, b).start()
        for j in range(max(span_rounds - 2, 0), span_rounds):
            c = j * NS + sid
            b = j % 2

            @pl.when(c < n_spans)
            def _():
                stage_b(c, b).wait()

    return sc_kernel(x, src, dst, w)


def _combine_body(p_ref, o_ref):
    o_ref[...] = p_ref[0] + p_ref[1]


def _combine(partials):
    nc, n, d = partials.shape
    blk = 1000
    return pl.pallas_call(
        _combine_body,
        grid=(n // blk,),
        in_specs=[pl.BlockSpec((nc, blk, d), lambda i: (0, i, 0))],
        out_specs=pl.BlockSpec((blk, d), lambda i: (i, 0)),
        out_shape=jax.ShapeDtypeStruct((n, d), jnp.float32),
    )(partials)


@jax.jit
def kernel(x, edge_index, edge_weights):
    src = edge_index[0]
    dst = edge_index[1]
    e = src.shape[0]
    e_pad = NC * NS * (-(-e // (NC * NS * CHUNK * NBUF)) * CHUNK * NBUF)
    pad = e_pad - e
    if pad:
        src = jnp.concatenate([src, jnp.zeros((pad,), src.dtype)])
        dst = jnp.concatenate([dst, jnp.zeros((pad,), dst.dtype)])
        edge_weights = jnp.concatenate(
            [edge_weights, jnp.zeros((pad,), edge_weights.dtype)])
    partials = _sc_partials(x, src, dst, edge_weights)
    return _combine(partials)


# speedup vs baseline: 1.8464x; 1.8464x over previous
import functools

import jax
import jax.numpy as jnp
from jax import lax
from jax.experimental import pallas as pl
from jax.experimental.pallas import tpu as pltpu
from jax.experimental.pallas import tpu_sc as plsc

NC = 2
NS = 16
LANES = 16
CHUNK = 40
NBUF = 6
WGRP = 8


def _sc_partials(x, src, dst, w):
    n, d = x.shape
    e = src.shape[0]
    nw = NC * NS
    e_per_w = e // nw
    assert e_per_w * nw == e and e_per_w % CHUNK == 0 and CHUNK % 8 == 0
    n_chunks = e_per_w // CHUNK
    assert n_chunks % NBUF == 0
    n_rounds = n_chunks // NBUF
    span = 40
    n_spans = n // span
    assert n % span == 0
    span_rounds = -(-n_spans // NS)
    nvec = d // LANES
    egrp = CHUNK // WGRP

    mesh = plsc.VectorSubcoreMesh(core_axis_name="c", subcore_axis_name="s")

    @functools.partial(
        pl.kernel,
        out_type=jax.ShapeDtypeStruct((NC, n, d), jnp.float32),
        mesh=mesh,
        scratch_types=[
            pltpu.VMEM((NBUF, CHUNK, d), jnp.float32),
            pltpu.VMEM((NBUF, CHUNK), jnp.int32),
            pltpu.VMEM((NBUF, CHUNK), jnp.int32),
            pltpu.VMEM((NBUF, CHUNK + LANES), jnp.float32),
            pltpu.VMEM_SHARED((n, d), jnp.float32),
            pltpu.SemaphoreType.DMA((NBUF,)),
            pltpu.SemaphoreType.DMA((NBUF,)),
            pltpu.SemaphoreType.DMA((NBUF,)),
            pltpu.SemaphoreType.DMA((NBUF,)),
            pltpu.SemaphoreType.DMA((NBUF,)),
        ],
    )
    def sc_kernel(x_hbm, src_hbm, dst_hbm, w_hbm, out_hbm, rows, srcb, dstb,
                  wb, acc, sem_g, sem_s, sem_src, sem_w, sem_d):
        cid = lax.axis_index("c")
        sid = lax.axis_index("s")
        wid = sid * NC + cid
        ebase = wid * e_per_w

        def src_copy(j, b):
            return pltpu.make_async_copy(
                src_hbm.at[pl.ds(ebase + j * CHUNK, CHUNK)], srcb.at[b],
                sem_src.at[b])

        def w_copy(j, b):
            return pltpu.make_async_copy(
                w_hbm.at[pl.ds(ebase + j * CHUNK, CHUNK)],
                wb.at[b, pl.ds(0, CHUNK)], sem_w.at[b])

        def dst_copy(j, b):
            return pltpu.make_async_copy(
                dst_hbm.at[pl.ds(ebase + j * CHUNK, CHUNK)], dstb.at[b],
                sem_d.at[b])

        def gather_copy(j, b):
            return pltpu.make_async_copy(
                x_hbm.at[srcb.at[b]], rows.at[b], sem_g.at[b])

        def scale_chunk(b):
            rows_b = rows.at[b]
            for g in range(egrp):
                w16 = wb[b, pl.ds(WGRP * g, LANES)]
                for ee in range(WGRP):
                    wsplat = jnp.full((LANES,), w16[ee])
                    i = WGRP * g + ee
                    for f in range(nvec):
                        sl = (i, pl.ds(LANES * f, LANES))
                        rows_b[sl] = rows_b[sl] * wsplat

        for c in range(4):
            src_copy(c, c).start()
            w_copy(c, c).start()
        for c in range(2):
            dst_copy(c, c).start()
        for c in range(2):
            src_copy(c, c).wait()
            gather_copy(c, c).start()

        zeros = jnp.zeros((LANES,), jnp.float32)

        def zero_row(i, carry):
            for f in range(nvec):
                rows[NBUF - 1, i, pl.ds(LANES * f, LANES)] = zeros
            return carry

        lax.fori_loop(0, span, zero_row, 0)

        for j in range(span_rounds):
            c = j * NS + sid

            @pl.when(c < n_spans)
            def _():
                pltpu.async_copy(rows.at[NBUF - 1, pl.ds(0, span)],
                                 acc.at[pl.ds(c * span, span)],
                                 sem_s.at[0])
        for j in range(span_rounds):
            c = j * NS + sid

            @pl.when(c < n_spans)
            def _():
                pltpu.make_async_copy(
                    rows.at[NBUF - 1, pl.ds(0, span)],
                    acc.at[pl.ds(c * span, span)], sem_s.at[0]).wait()
        plsc.subcore_barrier()

        def round_body(q, carry):
            for k in range(NBUF):
                j = q * NBUF + k
                b2 = (k + 2) % NBUF
                b4 = (k + 4) % NBUF

                @pl.when(j >= 4)
                def _():
                    pltpu.make_async_copy(
                        rows.at[b2], acc.at[dstb.at[b2]], sem_s.at[b2]).wait()

                @pl.when(j + 2 < n_chunks)
                def _():
                    dst_copy(j + 2, b2).start()
                    src_copy(j + 2, b2).wait()
                    gather_copy(j + 2, b2).start()

                @pl.when(j + 4 < n_chunks)
                def _():
                    src_copy(j + 4, b4).start()
                    w_copy(j + 4, b4).start()

                gather_copy(j, k).wait()
                w_copy(j, k).wait()
                scale_chunk(k)
                dst_copy(j, k).wait()
                pltpu.async_copy(rows.at[k], acc.at[dstb.at[k]], sem_s.at[k],
                                 add=True)
            return carry

        lax.fori_loop(0, n_rounds, round_body, 0)
        for c in range(n_chunks - 4, n_chunks):
            k = c % NBUF
            pltpu.make_async_copy(
                rows.at[k], acc.at[dstb.at[k]], sem_s.at[k]).wait()

        plsc.subcore_barrier()

        def stage_a(c, b):
            return pltpu.make_async_copy(
                acc.at[pl.ds(c * span, span)], rows.at[b, pl.ds(0, span)],
                sem_g.at[b])

        def stage_b(c, b):
            return pltpu.make_async_copy(
                rows.at[b, pl.ds(0, span)],
                out_hbm.at[cid, pl.ds(c * span, span)], sem_s.at[b])

        for j in range(span_rounds):
            c = j * NS + sid
            b = j % 2

            @pl.when(c < n_spans)
            def _():
                if j >= 2:
                    cprev = (j - 2) * NS + sid
                    stage_b(cprev, b).wait()
                stage_a(c, b).start()
                stage_a(c, b).wait()
                stage_b(c, b).start()
        for j in range(max(span_rounds - 2, 0), span_rounds):
            c = j * NS + sid
            b = j % 2

            @pl.when(c < n_spans)
            def _():
                stage_b(c, b).wait()

    return sc_kernel(x, src, dst, w)


def _combine_body(p_ref, o_ref):
    o_ref[...] = p_ref[0] + p_ref[1]


def _combine(partials):
    nc, n, d = partials.shape
    blk = 1000
    return pl.pallas_call(
        _combine_body,
        grid=(n // blk,),
        in_specs=[pl.BlockSpec((nc, blk, d), lambda i: (0, i, 0))],
        out_specs=pl.BlockSpec((blk, d), lambda i: (i, 0)),
        out_shape=jax.ShapeDtypeStruct((n, d), jnp.float32),
    )(partials)


@jax.jit
def kernel(x, edge_index, edge_weights):
    src = edge_index[0]
    dst = edge_index[1]
    e = src.shape[0]
    e_pad = NC * NS * (-(-e // (NC * NS * CHUNK * NBUF)) * CHUNK * NBUF)
    pad = e_pad - e
    if pad:
        src = jnp.concatenate([src, jnp.zeros((pad,), src.dtype)])
        dst = jnp.concatenate([dst, jnp.zeros((pad,), dst.dtype)])
        edge_weights = jnp.concatenate(
            [edge_weights, jnp.zeros((pad,), edge_weights.dtype)])
    partials = _sc_partials(x, src, dst, edge_weights)
    return _combine(partials)


# speedup vs baseline: 2.8162x; 1.5252x over previous
import functools

import jax
import jax.numpy as jnp
from jax import lax
from jax.experimental import pallas as pl
from jax.experimental.pallas import tpu as pltpu
from jax.experimental.pallas import tpu_sc as plsc

NC = 2
NS = 16
LANES = 16
CHUNK = 40
NBUF = 5
WGRP = 8


def _sc_partials(x, src, dst, w):
    n, d = x.shape
    e = src.shape[0]
    nw = NC * NS
    e_per_w = e // nw
    assert e_per_w * nw == e and e_per_w % CHUNK == 0 and CHUNK % 8 == 0
    n_chunks = e_per_w // CHUNK
    assert n_chunks % NBUF == 0
    n_rounds = n_chunks // NBUF
    n_spans = n // CHUNK
    assert n % CHUNK == 0
    span_rounds = -(-n_spans // NS)
    nvec = d // LANES
    egrp = CHUNK // WGRP
    wpad = CHUNK + LANES

    mesh = plsc.VectorSubcoreMesh(core_axis_name="c", subcore_axis_name="s")

    @functools.partial(
        pl.kernel,
        out_type=jax.ShapeDtypeStruct((NC, n, d), jnp.float32),
        mesh=mesh,
        scratch_types=[
            pltpu.VMEM((NBUF, CHUNK, d), jnp.float32),
            pltpu.VMEM((NBUF, CHUNK), jnp.int32),
            pltpu.VMEM((NBUF, CHUNK), jnp.int32),
            pltpu.VMEM((NBUF, wpad), jnp.float32),
            pltpu.VMEM_SHARED((n, d), jnp.float32),
            pltpu.SemaphoreType.DMA((NBUF,)),
            pltpu.SemaphoreType.DMA((NBUF,)),
            pltpu.SemaphoreType.DMA((NBUF,)),
            pltpu.SemaphoreType.DMA((NBUF,)),
            pltpu.SemaphoreType.DMA((NBUF,)),
        ],
    )
    def sc_kernel(x_hbm, src_hbm, dst_hbm, w_hbm, out_hbm, rows, srcb, dstb,
                  wb, acc, sem_g, sem_s, sem_sw, sem_w, sem_d):
        cid = lax.axis_index("c")
        sid = lax.axis_index("s")
        wid = sid * NC + cid
        ebase = wid * e_per_w

        def src_copy(j, b):
            return pltpu.make_async_copy(
                src_hbm.at[pl.ds(ebase + j * CHUNK, CHUNK)], srcb.at[b],
                sem_sw.at[b])

        def w_copy(j, b):
            return pltpu.make_async_copy(
                w_hbm.at[pl.ds(ebase + j * CHUNK, CHUNK)],
                wb.at[b, pl.ds(0, CHUNK)], sem_w.at[b])

        def dst_copy(j, b):
            return pltpu.make_async_copy(
                dst_hbm.at[pl.ds(ebase + j * CHUNK, CHUNK)], dstb.at[b],
                sem_d.at[b])

        def gather_copy(j, b):
            return pltpu.make_async_copy(
                x_hbm.at[srcb.at[b]], rows.at[b], sem_g.at[b])

        def scale_chunk(b):
            rows_b = rows.at[b]
            for g in range(egrp):
                w16 = wb[b, pl.ds(WGRP * g, LANES)]
                for ee in range(WGRP):
                    wsplat = jnp.full((LANES,), w16[ee])
                    i = WGRP * g + ee
                    for f in range(nvec):
                        sl = (i, pl.ds(LANES * f, LANES))
                        rows_b[sl] = rows_b[sl] * wsplat

        for c in range(2):
            src_copy(c, c).start()
            w_copy(c, c).start()
            dst_copy(c, c).start()
        for c in range(2, 4):
            src_copy(c, c).start()
            w_copy(c, c).start()

        zeros = jnp.zeros((LANES,), jnp.float32)

        def zero_row(i, carry):
            for f in range(nvec):
                rows[NBUF - 1, i, pl.ds(LANES * f, LANES)] = zeros
            return carry

        lax.fori_loop(0, CHUNK, zero_row, 0)

        for j in range(span_rounds):
            c = j * NS + sid

            @pl.when(c < n_spans)
            def _():
                pltpu.async_copy(rows.at[NBUF - 1],
                                 acc.at[pl.ds(c * CHUNK, CHUNK)],
                                 sem_s.at[0])
        for j in range(span_rounds):
            c = j * NS + sid

            @pl.when(c < n_spans)
            def _():
                pltpu.make_async_copy(
                    rows.at[NBUF - 1],
                    acc.at[pl.ds(c * CHUNK, CHUNK)], sem_s.at[0]).wait()
        plsc.subcore_barrier()

        for c in range(2):
            src_copy(c, c).wait()
            gather_copy(c, c).start()

        def round_body(q, carry):
            for k in range(NBUF):
                j = q * NBUF + k
                b2 = (k + 2) % NBUF
                b4 = (k + 4) % NBUF

                @pl.when(j >= 3)
                def _():
                    pltpu.make_async_copy(
                        rows.at[b2], acc.at[dstb.at[b2]], sem_s.at[b2]).wait()

                @pl.when(j + 2 < n_chunks)
                def _():
                    dst_copy(j + 2, b2).start()

                @pl.when(j + 4 < n_chunks)
                def _():
                    src_copy(j + 4, b4).start()
                    w_copy(j + 4, b4).start()

                @pl.when(j + 2 < n_chunks)
                def _():
                    src_copy(j + 2, b2).wait()
                    gather_copy(j + 2, b2).start()

                gather_copy(j, k).wait()
                w_copy(j, k).wait()
                scale_chunk(k)
                dst_copy(j, k).wait()
                pltpu.async_copy(rows.at[k], acc.at[dstb.at[k]], sem_s.at[k],
                                 add=True)
            return carry

        lax.fori_loop(0, n_rounds, round_body, 0)
        for c in range(n_chunks - 3, n_chunks):
            k = c % NBUF
            pltpu.make_async_copy(
                rows.at[k], acc.at[dstb.at[k]], sem_s.at[k]).wait()

        plsc.subcore_barrier()

        def stage_a(c, b):
            return pltpu.make_async_copy(
                acc.at[pl.ds(c * CHUNK, CHUNK)], rows.at[b], sem_g.at[b])

        def stage_b(c, b):
            return pltpu.make_async_copy(
                rows.at[b], out_hbm.at[cid, pl.ds(c * CHUNK, CHUNK)],
                sem_s.at[b])

        for j in range(span_rounds):
            c = j * NS + sid
            b = j % 2

            @pl.when(c < n_spans)
            def _():
                if j >= 2:
                    cprev = (j - 2) * NS + sid
                    stage_b(cprev, b).wait()
                stage_a(c, b).start()
                stage_a(c, b).wait()
                stage_b(c, b).start()
        for j in range(span_rounds - 2, span_rounds):
            c = j * NS + sid
            b = j % 2

            @pl.when(c < n_spans)
            def _():
                stage_b(c, b).wait()

    return sc_kernel(x, src, dst, w)


def _combine_body(p_ref, o_ref):
    o_ref[...] = p_ref[0] + p_ref[1]


def _combine(partials):
    nc, n, d = partials.shape
    blk = 1000
    return pl.pallas_call(
        _combine_body,
        grid=(n // blk,),
        in_specs=[pl.BlockSpec((nc, blk, d), lambda i: (0, i, 0))],
        out_specs=pl.BlockSpec((blk, d), lambda i: (i, 0)),
        out_shape=jax.ShapeDtypeStruct((n, d), jnp.float32),
    )(partials)


@jax.jit
def kernel(x, edge_index, edge_weights):
    src = edge_index[0]
    dst = edge_index[1]
    partials = _sc_partials(x, src, dst, edge_weights)
    return _combine(partials)
